# Initial kernel scaffold; baseline (speedup 1.0000x reference)
import jax
import jax.numpy as jnp
from jax.experimental import pallas as pl


def kernel(word_embeddings, pinyin_ids, glyph_ids, pos_ids, pinyin_table, glyph_table, tag_table, pos_table, fc_w, fc_b, ln_gamma, ln_beta):
    raise NotImplementedError("write your pallas kernel here")


# speedup vs baseline: 1.3021x; 1.3021x over previous
import functools

import jax
import jax.numpy as jnp
from jax import lax
from jax.experimental import pallas as pl
from jax.experimental.pallas import tpu as pltpu
from jax.experimental.pallas import tpu_sc as plsc

_HID = 768
_PD = 128
_GD = 576
_TD = 32
_GDP = 640
_TDP = 128
_EPS = 1e-12


def _sc_gather(pin_ids, gly_ids, tag_ids, pin_tab, gly_tab, tag_tab):
    nt = pin_ids.shape[0]
    info = plsc.get_sparse_core_info()
    nw = info.num_cores * info.num_subcores
    bpw = nt // nw
    ch = 64
    nch = bpw // ch
    mesh = plsc.VectorSubcoreMesh(core_axis_name="c", subcore_axis_name="s")

    @functools.partial(
        pl.kernel,
        mesh=mesh,
        out_type=(
            jax.ShapeDtypeStruct((nt, _PD), jnp.float32),
            jax.ShapeDtypeStruct((nt, _GDP), jnp.float32),
            jax.ShapeDtypeStruct((nt, _TDP), jnp.float32),
        ),
        scratch_types=[
            pltpu.VMEM((ch,), jnp.int32),
            pltpu.VMEM((ch,), jnp.int32),
            pltpu.VMEM((ch,), jnp.int32),
            pltpu.VMEM((ch, _PD), jnp.float32),
            pltpu.VMEM((ch, _GDP), jnp.float32),
            pltpu.VMEM((ch, _TDP), jnp.float32),
            pltpu.SemaphoreType.DMA,
        ],
    )
    def body(pin_ids_h, gly_ids_h, tag_ids_h, pin_tab_h, gly_tab_h, tag_tab_h,
             pin_out, gly_out, tag_out,
             pin_idx, gly_idx, tag_idx, pin_rows, gly_rows, tag_rows, sem):
        wid = lax.axis_index("s") * info.num_cores + lax.axis_index("c")
        base = wid * bpw

        def chunk(c, carry):
            off = base + c * ch
            pltpu.sync_copy(pin_ids_h.at[pl.ds(off, ch)], pin_idx)
            pltpu.sync_copy(gly_ids_h.at[pl.ds(off, ch)], gly_idx)
            pltpu.sync_copy(tag_ids_h.at[pl.ds(off, ch)], tag_idx)
            cp1 = pltpu.async_copy(pin_tab_h.at[pin_idx], pin_rows, sem)
            cp2 = pltpu.async_copy(gly_tab_h.at[gly_idx], gly_rows, sem)
            cp3 = pltpu.async_copy(tag_tab_h.at[tag_idx], tag_rows, sem)
            cp1.wait()
            cp2.wait()
            cp3.wait()
            pltpu.sync_copy(pin_rows, pin_out.at[pl.ds(off, ch)])
            pltpu.sync_copy(gly_rows, gly_out.at[pl.ds(off, ch)])
            pltpu.sync_copy(tag_rows, tag_out.at[pl.ds(off, ch)])
            return carry

        lax.fori_loop(0, nch, chunk, 0)

    return body(pin_ids, gly_ids, tag_ids, pin_tab, gly_tab, tag_tab)


def _tc_fused(word, pin_emb, gly_emb, tag_emb, w1, w2, w3, w4, bvec, pos,
              gamma, beta):
    m = word.shape[0]
    bm = 512
    grid = (m // bm,)

    def body(w_ref, p_ref, g_ref, t_ref, w1r, w2r, w3r, w4r, br, posr, gr,
             ber, out_ref):
        acc = jnp.dot(w_ref[...], w1r[...], preferred_element_type=jnp.float32)
        acc = acc + jnp.dot(p_ref[...], w2r[...],
                            preferred_element_type=jnp.float32)
        acc = acc + jnp.dot(g_ref[...], w3r[...],
                            preferred_element_type=jnp.float32)
        acc = acc + jnp.dot(t_ref[...], w4r[...],
                            preferred_element_type=jnp.float32)
        x = acc + br[...] + posr[...]
        mu = jnp.mean(x, axis=-1, keepdims=True)
        xc = x - mu
        var = jnp.mean(xc * xc, axis=-1, keepdims=True)
        out_ref[...] = xc * lax.rsqrt(var + _EPS) * gr[...] + ber[...]

    return pl.pallas_call(
        body,
        grid=grid,
        in_specs=[
            pl.BlockSpec((bm, _HID), lambda i: (i, 0)),
            pl.BlockSpec((bm, _PD), lambda i: (i, 0)),
            pl.BlockSpec((bm, _GDP), lambda i: (i, 0)),
            pl.BlockSpec((bm, _TDP), lambda i: (i, 0)),
            pl.BlockSpec((_HID, _HID), lambda i: (0, 0)),
            pl.BlockSpec((_PD, _HID), lambda i: (0, 0)),
            pl.BlockSpec((_GDP, _HID), lambda i: (0, 0)),
            pl.BlockSpec((_TDP, _HID), lambda i: (0, 0)),
            pl.BlockSpec((1, _HID), lambda i: (0, 0)),
            pl.BlockSpec((bm, _HID), lambda i: (0, 0)),
            pl.BlockSpec((1, _HID), lambda i: (0, 0)),
            pl.BlockSpec((1, _HID), lambda i: (0, 0)),
        ],
        out_specs=pl.BlockSpec((bm, _HID), lambda i: (i, 0)),
        out_shape=jax.ShapeDtypeStruct((m, _HID), jnp.float32),
    )(word, pin_emb, gly_emb, tag_emb, w1, w2, w3, w4, bvec, pos, gamma, beta)


def kernel(word_embeddings, pinyin_ids, glyph_ids, pos_ids, pinyin_table,
           glyph_table, tag_table, pos_table, fc_w, fc_b, ln_gamma, ln_beta):
    b, l, h = word_embeddings.shape
    nt = b * l
    pin_ids = pinyin_ids.reshape(nt).astype(jnp.int32)
    gly_ids = glyph_ids.reshape(nt).astype(jnp.int32)
    tag_ids = pos_ids.reshape(nt).astype(jnp.int32)

    gly_tab_p = jnp.pad(glyph_table, ((0, 0), (0, _GDP - _GD)))
    tag_tab_p = jnp.pad(tag_table, ((0, 0), (0, _TDP - _TD)))

    pin_emb, gly_emb, tag_emb = _sc_gather(
        pin_ids, gly_ids, tag_ids, pinyin_table, gly_tab_p, tag_tab_p)

    w1 = fc_w[:h]
    w2 = fc_w[h:h + _PD]
    w3 = jnp.pad(fc_w[h + _PD:h + _PD + _GD], ((0, _GDP - _GD), (0, 0)))
    w4 = jnp.pad(fc_w[h + _PD + _GD:], ((0, _TDP - _TD), (0, 0)))

    out = _tc_fused(
        word_embeddings.reshape(nt, h), pin_emb, gly_emb, tag_emb,
        w1, w2, w3, w4,
        fc_b.reshape(1, h), pos_table,
        ln_gamma.reshape(1, h), ln_beta.reshape(1, h))
    return out.reshape(b, l, h)
